# Initial kernel scaffold; baseline (speedup 1.0000x reference)
import jax
import jax.numpy as jnp
from jax.experimental import pallas as pl


def kernel(tokens, emb, W1, b1, W2, b2, W3, b3):
    raise NotImplementedError("write your pallas kernel here")


# speedup vs baseline: 83.3992x; 83.3992x over previous
import functools

import jax
import jax.numpy as jnp
from jax import lax
from jax.experimental import pallas as pl
from jax.experimental.pallas import tpu as pltpu
from jax.experimental.pallas import tpu_sc as plsc

B = 16384
S = 200
VOCAB = 95
VOC = 96
EMBED = 32
HIDDEN = 64
LATENT = 32

NC = 2
NS = 16
NW = NC * NS
RPW = B // NW
CH = 256
NCHUNK = RPW // CH

_mesh = plsc.VectorSubcoreMesh(core_axis_name="c", subcore_axis_name="s")


@functools.partial(
    pl.kernel,
    mesh=_mesh,
    out_type=jax.ShapeDtypeStruct((B, VOC), jnp.float32),
    scratch_types=[
        pltpu.VMEM((CH, S), jnp.int32),
        pltpu.VMEM((RPW, VOC), jnp.float32),
    ],
    compiler_params=pltpu.CompilerParams(needs_layout_passes=False),
)
def _sc_histogram(tok_hbm, cnt_hbm, tok_v, cnt_v):
    wid = lax.axis_index("s") * NC + lax.axis_index("c")
    row0 = wid * RPW
    lanes = lax.iota(jnp.int32, 16)
    zeros = jnp.zeros((16,), jnp.float32)
    ones = jnp.ones((16,), jnp.float32)

    def zbody(i, carry):
        for c in range(VOC // 16):
            cnt_v[i, pl.ds(c * 16, 16)] = zeros
        return carry

    lax.fori_loop(0, RPW, zbody, 0, unroll=4)

    for ch in range(NCHUNK):
        pltpu.sync_copy(tok_hbm.at[pl.ds(row0 + ch * CH, CH)], tok_v)
        for rb in range(CH // 16):
            tok_rows = rb * 16 + lanes
            cnt_rows = (ch * CH + rb * 16) + lanes

            def sbody(s, carry):
                svec = jnp.full((16,), 0, jnp.int32) + s
                t = plsc.load_gather(tok_v, [tok_rows, svec])
                plsc.addupdate_scatter(cnt_v, [cnt_rows, t], ones)
                return carry

            lax.fori_loop(0, S, sbody, 0, unroll=4)

    pltpu.sync_copy(cnt_v, cnt_hbm.at[pl.ds(row0, RPW)])


def _tc_mlp(cnt_ref, emb_ref, w1_ref, b1_ref, w2_ref, b2_ref, w3_ref, b3_ref,
            out_ref):
    counts = cnt_ref[...]
    e1 = jnp.dot(emb_ref[...], w1_ref[...],
                 preferred_element_type=jnp.float32)
    acc = jnp.dot(counts, e1, preferred_element_type=jnp.float32)
    denom = (jnp.float32(S) - counts[:, 0:1]) + jnp.float32(1e-8)
    h = jnp.maximum(acc / denom + b1_ref[...], 0.0)
    h = jnp.maximum(
        jnp.dot(h, w2_ref[...], preferred_element_type=jnp.float32)
        + b2_ref[...], 0.0)
    out_ref[...] = (jnp.dot(h, w3_ref[...], preferred_element_type=jnp.float32)
                    + b3_ref[...])


BLK = 2048


def kernel(tokens, emb, W1, b1, W2, b2, W3, b3):
    tokens = tokens.astype(jnp.int32)
    counts = _sc_histogram(tokens)
    emb96 = jnp.pad(emb.astype(jnp.float32), ((0, VOC - VOCAB), (0, 0)))
    rep = lambda i: (0, 0)
    z = pl.pallas_call(
        _tc_mlp,
        grid=(B // BLK,),
        in_specs=[
            pl.BlockSpec((BLK, VOC), lambda i: (i, 0)),
            pl.BlockSpec((VOC, EMBED), rep),
            pl.BlockSpec((EMBED, HIDDEN), rep),
            pl.BlockSpec((1, HIDDEN), rep),
            pl.BlockSpec((HIDDEN, HIDDEN), rep),
            pl.BlockSpec((1, HIDDEN), rep),
            pl.BlockSpec((HIDDEN, LATENT), rep),
            pl.BlockSpec((1, LATENT), rep),
        ],
        out_specs=pl.BlockSpec((BLK, LATENT), lambda i: (i, 0)),
        out_shape=jax.ShapeDtypeStruct((B, LATENT), jnp.float32),
    )(counts, emb96, W1, b1.reshape(1, -1), W2, b2.reshape(1, -1),
      W3, b3.reshape(1, -1))
    return z


# speedup vs baseline: 119.5874x; 1.4339x over previous
import functools

import jax
import jax.numpy as jnp
from jax import lax
from jax.experimental import pallas as pl
from jax.experimental.pallas import tpu as pltpu
from jax.experimental.pallas import tpu_sc as plsc

B = 16384
S = 200
VOCAB = 95
VOC = 96
EMBED = 32
HIDDEN = 64
LATENT = 32

NC = 2
NS = 16
NW = NC * NS
RPW = B // NW
CH = 128
NCHUNK = RPW // CH

_mesh = plsc.VectorSubcoreMesh(core_axis_name="c", subcore_axis_name="s")


@functools.partial(
    pl.kernel,
    mesh=_mesh,
    out_type=jax.ShapeDtypeStruct((B, VOC), jnp.float32),
    scratch_types=[
        pltpu.VMEM((CH, S), jnp.int32),
        pltpu.VMEM((CH, S), jnp.int32),
        pltpu.VMEM((RPW, VOC), jnp.float32),
        pltpu.SemaphoreType.DMA,
        pltpu.SemaphoreType.DMA,
        pltpu.SemaphoreType.DMA,
    ],
    compiler_params=pltpu.CompilerParams(needs_layout_passes=False),
)
def _sc_histogram(tok_hbm, cnt_hbm, tok_v0, tok_v1, cnt_v, sem0, sem1,
                  sem_out):
    wid = lax.axis_index("s") * NC + lax.axis_index("c")
    row0 = wid * RPW
    lanes = lax.iota(jnp.int32, 16)
    zeros = jnp.zeros((16,), jnp.float32)
    ones = jnp.ones((16,), jnp.float32)
    bufs = (tok_v0, tok_v1)
    sems = (sem0, sem1)

    in_descs = [None] * NCHUNK
    in_descs[0] = pltpu.async_copy(tok_hbm.at[pl.ds(row0, CH)], tok_v0, sem0)

    def zbody(i, carry):
        for c in range(VOC // 16):
            cnt_v[i, pl.ds(c * 16, 16)] = zeros
        return carry

    lax.fori_loop(0, RPW, zbody, 0, unroll=4)

    out_descs = []
    for ch in range(NCHUNK):
        if ch + 1 < NCHUNK:
            in_descs[ch + 1] = pltpu.async_copy(
                tok_hbm.at[pl.ds(row0 + (ch + 1) * CH, CH)],
                bufs[(ch + 1) % 2], sems[(ch + 1) % 2])
        in_descs[ch].wait()
        tok_v = bufs[ch % 2]
        row_groups = [(ch * CH + rb * 16) + lanes for rb in range(CH // 16)]
        loc_groups = [rb * 16 + lanes for rb in range(CH // 16)]

        def sbody(s, carry):
            svec = jnp.full((16,), 0, jnp.int32) + s
            ts = [plsc.load_gather(tok_v, [loc_groups[rb], svec])
                  for rb in range(CH // 16)]
            for rb in range(CH // 16):
                plsc.addupdate_scatter(cnt_v, [row_groups[rb], ts[rb]], ones)
            return carry

        lax.fori_loop(0, S, sbody, 0, unroll=2)
        out_descs.append(pltpu.async_copy(
            cnt_v.at[pl.ds(ch * CH, CH)],
            cnt_hbm.at[pl.ds(row0 + ch * CH, CH)], sem_out))

    for d in out_descs:
        d.wait()


def _tc_mlp(cnt_ref, emb_ref, w1_ref, b1_ref, w2_ref, b2_ref, w3_ref, b3_ref,
            out_ref):
    counts = cnt_ref[...]
    e1 = jnp.dot(emb_ref[...], w1_ref[...],
                 preferred_element_type=jnp.float32)
    acc = jnp.dot(counts, e1, preferred_element_type=jnp.float32)
    denom = (jnp.float32(S) - counts[:, 0:1]) + jnp.float32(1e-8)
    h = jnp.maximum(acc / denom + b1_ref[...], 0.0)
    h = jnp.maximum(
        jnp.dot(h, w2_ref[...], preferred_element_type=jnp.float32)
        + b2_ref[...], 0.0)
    out_ref[...] = (jnp.dot(h, w3_ref[...], preferred_element_type=jnp.float32)
                    + b3_ref[...])


BLK = 2048


def kernel(tokens, emb, W1, b1, W2, b2, W3, b3):
    tokens = tokens.astype(jnp.int32)
    counts = _sc_histogram(tokens)
    emb96 = jnp.pad(emb.astype(jnp.float32), ((0, VOC - VOCAB), (0, 0)))
    rep = lambda i: (0, 0)
    z = pl.pallas_call(
        _tc_mlp,
        grid=(B // BLK,),
        in_specs=[
            pl.BlockSpec((BLK, VOC), lambda i: (i, 0)),
            pl.BlockSpec((VOC, EMBED), rep),
            pl.BlockSpec((EMBED, HIDDEN), rep),
            pl.BlockSpec((1, HIDDEN), rep),
            pl.BlockSpec((HIDDEN, HIDDEN), rep),
            pl.BlockSpec((1, HIDDEN), rep),
            pl.BlockSpec((HIDDEN, LATENT), rep),
            pl.BlockSpec((1, LATENT), rep),
        ],
        out_specs=pl.BlockSpec((BLK, LATENT), lambda i: (i, 0)),
        out_shape=jax.ShapeDtypeStruct((B, LATENT), jnp.float32),
    )(counts, emb96, W1, b1.reshape(1, -1), W2, b2.reshape(1, -1),
      W3, b3.reshape(1, -1))
    return z


# speedup vs baseline: 213.3884x; 1.7844x over previous
import functools

import jax
import jax.numpy as jnp
from jax import lax
from jax.experimental import pallas as pl
from jax.experimental.pallas import tpu as pltpu
from jax.experimental.pallas import tpu_sc as plsc

B = 16384
S = 200
VOCAB = 95
VOC = 96
EMBED = 32
HIDDEN = 64
LATENT = 32

NC = 2
NS = 16
NW = NC * NS
RPW = B // NW
CH = 128
NCHUNK = RPW // CH

_mesh = plsc.VectorSubcoreMesh(core_axis_name="c", subcore_axis_name="s")


@functools.partial(
    pl.kernel,
    mesh=_mesh,
    out_type=jax.ShapeDtypeStruct((B, VOC), jnp.float32),
    scratch_types=[
        pltpu.VMEM((CH, S), jnp.int32),
        pltpu.VMEM((CH, S), jnp.int32),
        pltpu.VMEM((RPW, VOC), jnp.float32),
        pltpu.SemaphoreType.DMA,
        pltpu.SemaphoreType.DMA,
        pltpu.SemaphoreType.DMA,
    ],
    compiler_params=pltpu.CompilerParams(needs_layout_passes=False),
)
def _sc_histogram(tok_hbm, cnt_hbm, tok_v0, tok_v1, cnt_v, sem0, sem1,
                  sem_out):
    wid = lax.axis_index("s") * NC + lax.axis_index("c")
    row0 = wid * RPW
    lanes = lax.iota(jnp.int32, 16)
    zeros = jnp.zeros((16,), jnp.float32)
    ones = jnp.ones((16,), jnp.float32)
    bufs = (tok_v0, tok_v1)
    sems = (sem0, sem1)

    in_descs = [None] * NCHUNK
    in_descs[0] = pltpu.async_copy(tok_hbm.at[pl.ds(row0, CH)], tok_v0, sem0)

    def zbody(i, carry):
        for c in range(VOC // 16):
            cnt_v[i, pl.ds(c * 16, 16)] = zeros
        return carry

    lax.fori_loop(0, RPW, zbody, 0, unroll=4)

    out_descs = []
    for ch in range(NCHUNK):
        if ch + 1 < NCHUNK:
            in_descs[ch + 1] = pltpu.async_copy(
                tok_hbm.at[pl.ds(row0 + (ch + 1) * CH, CH)],
                bufs[(ch + 1) % 2], sems[(ch + 1) % 2])
        in_descs[ch].wait()
        tok_v = bufs[ch % 2]
        row_groups = [(ch * CH + rb * 16) + lanes for rb in range(CH // 16)]
        loc_groups = [rb * 16 + lanes for rb in range(CH // 16)]

        def sbody(s, carry):
            sv = lanes + s
            svec = jnp.where(sv >= S, sv - S, sv)
            ts = [plsc.load_gather(tok_v, [loc_groups[rb], svec])
                  for rb in range(CH // 16)]
            for rb in range(CH // 16):
                plsc.addupdate_scatter(cnt_v, [row_groups[rb], ts[rb]], ones)
            return carry

        lax.fori_loop(0, S, sbody, 0, unroll=2)
        out_descs.append(pltpu.async_copy(
            cnt_v.at[pl.ds(ch * CH, CH)],
            cnt_hbm.at[pl.ds(row0 + ch * CH, CH)], sem_out))

    for d in out_descs:
        d.wait()


def _tc_mlp(cnt_ref, emb_ref, w1_ref, b1_ref, w2_ref, b2_ref, w3_ref, b3_ref,
            out_ref):
    counts = cnt_ref[...]
    e1 = jnp.dot(emb_ref[...], w1_ref[...],
                 preferred_element_type=jnp.float32)
    acc = jnp.dot(counts, e1, preferred_element_type=jnp.float32)
    denom = (jnp.float32(S) - counts[:, 0:1]) + jnp.float32(1e-8)
    h = jnp.maximum(acc / denom + b1_ref[...], 0.0)
    h = jnp.maximum(
        jnp.dot(h, w2_ref[...], preferred_element_type=jnp.float32)
        + b2_ref[...], 0.0)
    out_ref[...] = (jnp.dot(h, w3_ref[...], preferred_element_type=jnp.float32)
                    + b3_ref[...])


BLK = 2048


def kernel(tokens, emb, W1, b1, W2, b2, W3, b3):
    tokens = tokens.astype(jnp.int32)
    counts = _sc_histogram(tokens)
    emb96 = jnp.pad(emb.astype(jnp.float32), ((0, VOC - VOCAB), (0, 0)))
    rep = lambda i: (0, 0)
    z = pl.pallas_call(
        _tc_mlp,
        grid=(B // BLK,),
        in_specs=[
            pl.BlockSpec((BLK, VOC), lambda i: (i, 0)),
            pl.BlockSpec((VOC, EMBED), rep),
            pl.BlockSpec((EMBED, HIDDEN), rep),
            pl.BlockSpec((1, HIDDEN), rep),
            pl.BlockSpec((HIDDEN, HIDDEN), rep),
            pl.BlockSpec((1, HIDDEN), rep),
            pl.BlockSpec((HIDDEN, LATENT), rep),
            pl.BlockSpec((1, LATENT), rep),
        ],
        out_specs=pl.BlockSpec((BLK, LATENT), lambda i: (i, 0)),
        out_shape=jax.ShapeDtypeStruct((B, LATENT), jnp.float32),
    )(counts, emb96, W1, b1.reshape(1, -1), W2, b2.reshape(1, -1),
      W3, b3.reshape(1, -1))
    return z


# speedup vs baseline: 213.5504x; 1.0008x over previous
import functools

import jax
import jax.numpy as jnp
from jax import lax
from jax.experimental import pallas as pl
from jax.experimental.pallas import tpu as pltpu
from jax.experimental.pallas import tpu_sc as plsc

B = 16384
S = 200
VOCAB = 95
VOC = 96
EMBED = 32
HIDDEN = 64
LATENT = 32

NC = 2
NS = 16
NW = NC * NS
RPW = B // NW
CH = 128
NCHUNK = RPW // CH

_mesh = plsc.VectorSubcoreMesh(core_axis_name="c", subcore_axis_name="s")


@functools.partial(
    pl.kernel,
    mesh=_mesh,
    out_type=jax.ShapeDtypeStruct((B, VOC), jnp.float32),
    scratch_types=[
        pltpu.VMEM((CH, S), jnp.int32),
        pltpu.VMEM((CH, S), jnp.int32),
        pltpu.VMEM((RPW, VOC), jnp.float32),
        pltpu.SemaphoreType.DMA,
        pltpu.SemaphoreType.DMA,
        pltpu.SemaphoreType.DMA,
    ],
    compiler_params=pltpu.CompilerParams(needs_layout_passes=False,
                                         use_tc_tiling_on_sc=True),
)
def _sc_histogram(tok_hbm, cnt_hbm, tok_v0, tok_v1, cnt_v, sem0, sem1,
                  sem_out):
    wid = lax.axis_index("s") * NC + lax.axis_index("c")
    row0 = wid * RPW
    lanes = lax.iota(jnp.int32, 16)
    zeros = jnp.zeros((16,), jnp.float32)
    ones = jnp.ones((16,), jnp.float32)
    bufs = (tok_v0, tok_v1)
    sems = (sem0, sem1)

    in_descs = [None] * NCHUNK
    in_descs[0] = pltpu.async_copy(tok_hbm.at[pl.ds(row0, CH)], tok_v0, sem0)

    def zbody(i, carry):
        for c in range(VOC // 16):
            cnt_v[i, pl.ds(c * 16, 16)] = zeros
        return carry

    lax.fori_loop(0, RPW, zbody, 0, unroll=4)

    out_descs = []
    for ch in range(NCHUNK):
        if ch + 1 < NCHUNK:
            in_descs[ch + 1] = pltpu.async_copy(
                tok_hbm.at[pl.ds(row0 + (ch + 1) * CH, CH)],
                bufs[(ch + 1) % 2], sems[(ch + 1) % 2])
        in_descs[ch].wait()
        tok_v = bufs[ch % 2]
        row_groups = [(ch * CH + rb * 16) + lanes for rb in range(CH // 16)]
        loc_groups = [rb * 16 + lanes for rb in range(CH // 16)]

        def sbody(s, carry):
            sv = lanes + s
            svec = jnp.where(sv >= S, sv - S, sv)
            ts = [plsc.load_gather(tok_v, [loc_groups[rb], svec])
                  for rb in range(CH // 16)]
            for rb in range(CH // 16):
                plsc.addupdate_scatter(cnt_v, [row_groups[rb], ts[rb]], ones)
            return carry

        lax.fori_loop(0, S, sbody, 0, unroll=2)
        out_descs.append(pltpu.async_copy(
            cnt_v.at[pl.ds(ch * CH, CH)],
            cnt_hbm.at[pl.ds(row0 + ch * CH, CH)], sem_out))

    for d in out_descs:
        d.wait()


def _tc_mlp(cnt_ref, emb_ref, w1_ref, b1_ref, w2_ref, b2_ref, w3_ref, b3_ref,
            out_ref):
    counts = cnt_ref[...]
    e1 = jnp.dot(emb_ref[...], w1_ref[...],
                 preferred_element_type=jnp.float32)
    acc = jnp.dot(counts, e1, preferred_element_type=jnp.float32)
    denom = (jnp.float32(S) - counts[:, 0:1]) + jnp.float32(1e-8)
    h = jnp.maximum(acc / denom + b1_ref[...], 0.0)
    h = jnp.maximum(
        jnp.dot(h, w2_ref[...], preferred_element_type=jnp.float32)
        + b2_ref[...], 0.0)
    out_ref[...] = (jnp.dot(h, w3_ref[...], preferred_element_type=jnp.float32)
                    + b3_ref[...])


BLK = 2048


def kernel(tokens, emb, W1, b1, W2, b2, W3, b3):
    tokens = tokens.astype(jnp.int32)
    counts = _sc_histogram(tokens)
    emb96 = jnp.pad(emb.astype(jnp.float32), ((0, VOC - VOCAB), (0, 0)))
    rep = lambda i: (0, 0)
    z = pl.pallas_call(
        _tc_mlp,
        grid=(B // BLK,),
        in_specs=[
            pl.BlockSpec((BLK, VOC), lambda i: (i, 0)),
            pl.BlockSpec((VOC, EMBED), rep),
            pl.BlockSpec((EMBED, HIDDEN), rep),
            pl.BlockSpec((1, HIDDEN), rep),
            pl.BlockSpec((HIDDEN, HIDDEN), rep),
            pl.BlockSpec((1, HIDDEN), rep),
            pl.BlockSpec((HIDDEN, LATENT), rep),
            pl.BlockSpec((1, LATENT), rep),
        ],
        out_specs=pl.BlockSpec((BLK, LATENT), lambda i: (i, 0)),
        out_shape=jax.ShapeDtypeStruct((B, LATENT), jnp.float32),
    )(counts, emb96, W1, b1.reshape(1, -1), W2, b2.reshape(1, -1),
      W3, b3.reshape(1, -1))
    return z
